# baseline (speedup 1.0000x reference)
import functools

import jax
import jax.numpy as jnp
from jax import lax
from jax.experimental import pallas as pl
from jax.experimental.pallas import tpu as pltpu
from jax.experimental.pallas import tpu_sc as plsc

QK_DIM = 32
TOPK = 16
N = 2048
BATCH = 8
BQ = 256
NSHARD = 4
SHARD = N // NSHARD
NCAND = NSHARD * TOPK
ROWS = BATCH * N


def _tc_shard_topk_kernel(q_ref, k_ref, val_ref, idx_ref):
    scale = QK_DIM ** (-0.5)
    q = q_ref[0] * jnp.float32(scale)
    k = k_ref[0]
    logits = jax.lax.dot_general(
        q, k, (((1,), (1,)), ((), ())),
        preferred_element_type=jnp.float32)
    neg = jnp.float32(-jnp.inf)
    vouts, iouts = [], []
    for s in range(NSHARD):
        sh = logits[:, s * SHARD:(s + 1) * SHARD]
        col = jax.lax.broadcasted_iota(jnp.int32, sh.shape, 1)
        for r in range(TOPK):
            m = jnp.max(sh, axis=1, keepdims=True)
            idx = jnp.argmax(sh, axis=1, keepdims=True)
            vouts.append(m)
            iouts.append(idx + s * SHARD)
            if r < TOPK - 1:
                sh = jnp.where(col == idx, neg, sh)
    val_ref[0] = jnp.concatenate(vouts, axis=1)
    idx_ref[0] = jnp.concatenate(iouts, axis=1)


def _tc_stage(query, key):
    grid = (BATCH, N // BQ)
    return pl.pallas_call(
        _tc_shard_topk_kernel,
        grid=grid,
        in_specs=[
            pl.BlockSpec((1, BQ, QK_DIM), lambda b, i: (b, i, 0)),
            pl.BlockSpec((1, N, QK_DIM), lambda b, i: (b, 0, 0)),
        ],
        out_specs=[
            pl.BlockSpec((1, BQ, NCAND), lambda b, i: (b, i, 0)),
            pl.BlockSpec((1, BQ, NCAND), lambda b, i: (b, i, 0)),
        ],
        out_shape=[
            jax.ShapeDtypeStruct((BATCH, N, NCAND), jnp.float32),
            jax.ShapeDtypeStruct((BATCH, N, NCAND), jnp.int32),
        ],
        compiler_params=pltpu.CompilerParams(
            dimension_semantics=("parallel", "parallel")),
    )(query, key)


def _merge16(av, ai, bv, bi):
    rbv = lax.rev(bv, (0,))
    rbi = lax.rev(bi, (0,))
    ge = av >= rbv
    cv = jnp.where(ge, av, rbv)
    ci = jnp.where(ge, ai, rbi)
    cv, ci = plsc.sort_key_val(cv, ci, descending=True)
    return cv, ci


def _make_sc_merge():
    info = plsc.get_sparse_core_info()
    nw = info.num_cores * info.num_subcores
    rpw = ROWS // nw
    mesh = plsc.VectorSubcoreMesh(core_axis_name="c", subcore_axis_name="s")

    chunk = 128

    @functools.partial(
        pl.kernel, mesh=mesh,
        out_type=jax.ShapeDtypeStruct((ROWS, TOPK), jnp.int32),
        compiler_params=pltpu.CompilerParams(needs_layout_passes=False),
        scratch_types=[
            pltpu.VMEM((chunk, NCAND), jnp.float32),
            pltpu.VMEM((chunk, NCAND), jnp.int32),
            pltpu.VMEM((chunk, TOPK), jnp.int32),
        ],
    )
    def _sc_merge(vals_hbm, idx_hbm, out_hbm, vals_v, idx_v, out_v):
        wid = lax.axis_index("s") * info.num_cores + lax.axis_index("c")
        base = wid * rpw

        def body(r, carry):
            lists = []
            for s in range(NSHARD):
                lists.append((vals_v[r, pl.ds(s * TOPK, TOPK)],
                              idx_v[r, pl.ds(s * TOPK, TOPK)]))
            (v0, i0), (v1, i1), (v2, i2), (v3, i3) = lists
            va, ia = _merge16(v0, i0, v1, i1)
            vb, ib = _merge16(v2, i2, v3, i3)
            _, iw = _merge16(va, ia, vb, ib)
            out_v[r, :] = iw
            return carry

        for c in range(rpw // chunk):
            cbase = base + c * chunk
            pltpu.sync_copy(vals_hbm.at[pl.ds(cbase, chunk)], vals_v)
            pltpu.sync_copy(idx_hbm.at[pl.ds(cbase, chunk)], idx_v)
            lax.fori_loop(0, chunk, body, 0)
            pltpu.sync_copy(out_v, out_hbm.at[pl.ds(cbase, chunk)])

    return _sc_merge


def kernel(query, key):
    cand_v, cand_i = _tc_stage(query, key)
    cv = cand_v.reshape(ROWS, NCAND)
    ci = cand_i.reshape(ROWS, NCAND)
    out = _make_sc_merge()(cv, ci)
    return out.reshape(BATCH, N, TOPK)


# speedup vs baseline: 7.7629x; 7.7629x over previous
import jax
import jax.numpy as jnp
from jax.experimental import pallas as pl
from jax.experimental.pallas import tpu as pltpu

QK_DIM = 32
TOPK = 16
N = 2048
BQ = 256


def _topk_route_kernel(q_ref, k_ref, out_ref):
    scale = QK_DIM ** (-0.5)
    q = q_ref[0] * jnp.float32(scale)
    k = k_ref[0]
    logits = jax.lax.dot_general(
        q, k, (((1,), (1,)), ((), ())),
        preferred_element_type=jnp.float32)

    col = jax.lax.broadcasted_iota(jnp.int32, logits.shape, 1)
    neg = jnp.float32(-jnp.inf)
    outs = []
    for r in range(TOPK):
        idx = jnp.argmax(logits, axis=1, keepdims=True)
        outs.append(idx)
        if r < TOPK - 1:
            logits = jnp.where(col == idx, neg, logits)

    out_ref[0] = jnp.concatenate(outs, axis=1)


def _topk_call(query, key):
    batch = query.shape[0]
    grid = (batch, N // BQ)
    return pl.pallas_call(
        _topk_route_kernel,
        grid=grid,
        in_specs=[
            pl.BlockSpec((1, BQ, QK_DIM), lambda b, i: (b, i, 0)),
            pl.BlockSpec((1, N, QK_DIM), lambda b, i: (b, 0, 0)),
        ],
        out_specs=pl.BlockSpec((1, BQ, TOPK), lambda b, i: (b, i, 0)),
        out_shape=jax.ShapeDtypeStruct((batch, N, TOPK), jnp.int32),
        compiler_params=pltpu.CompilerParams(
            dimension_semantics=("parallel", "parallel")),
    )(query, key)


def kernel(query, key):
    return _topk_call(query, key)
